# baseline (speedup 1.0000x reference)
import functools

import jax
import jax.numpy as jnp
from jax import lax
from jax.experimental import pallas as pl
from jax.experimental.pallas import tpu as pltpu
from jax.experimental.pallas import tpu_sc as plsc

NNODE = 10000
NP = 10240
NEDGE = 320000
NFE = 128
NCL = 64
NC, NS, LANES = 2, 16, 16
CH = 32
NCHW = 320
EPW = NCHW * CH
EPAD = NC * NS * EPW
NBUF = 8
LOOK = 6
NSTG = 8
SCH = NCHW // NSTG
RPT = NP // NS
ROWB = 512
GRID = NP // ROWB

def _mesh():
    return plsc.VectorSubcoreMesh(
        core_axis_name="c", subcore_axis_name="s", num_cores=NC, num_subcores=NS)


def _deg_body(ridx_hbm, out_hbm, ridx_v, ones_v, stage_v, acc):
    c = lax.axis_index("c")
    s = lax.axis_index("s")
    for q in range(CH // LANES):
        ones_v[pl.ds(q * LANES, LANES)] = jnp.ones((LANES,), jnp.float32)
    for q in range(RPT // LANES):
        stage_v[pl.ds(q * LANES, LANES)] = jnp.zeros((LANES,), jnp.float32)
    pltpu.sync_copy(ridx_hbm.at[c, s], ridx_v)
    pltpu.sync_copy(stage_v, acc.at[pl.ds(s * RPT, RPT)])
    plsc.subcore_barrier()

    def chunk(j, carry):
        pltpu.sync_copy(ones_v, acc.at[ridx_v.at[j]], add=True)
        return carry

    lax.fori_loop(0, NCHW, chunk, 0)
    plsc.subcore_barrier()
    pltpu.sync_copy(acc.at[pl.ds(s * RPT, RPT)], stage_v)
    pltpu.sync_copy(stage_v, out_hbm.at[c, pl.ds(s * RPT, RPT)])


def _deg(ridx):
    return pl.kernel(
        _deg_body,
        out_type=jax.ShapeDtypeStruct((NC, NP), jnp.float32),
        mesh=_mesh(),
        scratch_types=[
            pltpu.VMEM((NCHW, CH), jnp.int32),
            pltpu.VMEM((CH,), jnp.float32),
            pltpu.VMEM((RPT,), jnp.float32),
            pltpu.VMEM_SHARED((NP,), jnp.float32),
        ],
    )(ridx)


def _agg_body(g_hbm, ridx_hbm, cidx_hbm, out_hbm,
              ridx_v, cidx_v, b0, b1, b2, b3, b4, b5, b6, b7, acc,
              g0, g1, g2, g3, g4, g5, g6, g7,
              s0, s1, s2, s3, s4, s5, s6, s7):
    bufs = (b0, b1, b2, b3, b4, b5, b6, b7)
    gsems = (g0, g1, g2, g3, g4, g5, g6, g7)
    ssems = (s0, s1, s2, s3, s4, s5, s6, s7)
    c = lax.axis_index("c")
    s = lax.axis_index("s")
    for r in range(LANES):
        for q in range(NFE // LANES):
            b0[r, pl.ds(q * LANES, LANES)] = jnp.zeros((LANES,), jnp.float32)
    for k in range(RPT // LANES):
        pltpu.sync_copy(b0.at[pl.ds(0, LANES)],
                        acc.at[pl.ds(s * RPT + k * LANES, LANES)])
    plsc.subcore_barrier()

    for h in range(NSTG):
        pltpu.sync_copy(ridx_hbm.at[c, s, pl.ds(h * SCH, SCH)], ridx_v)
        pltpu.sync_copy(cidx_hbm.at[c, s, pl.ds(h * SCH, SCH)], cidx_v)
        for b in range(LOOK):
            pltpu.async_copy(g_hbm.at[ridx_v.at[b]], bufs[b], gsems[b])

        def inner(i, carry):
            for u in range(NBUF):
                j = i * NBUF + u
                b = u
                pltpu.make_async_copy(
                    g_hbm.at[ridx_v.at[j]], bufs[b], gsems[b]).wait()
                pltpu.async_copy(bufs[b], acc.at[cidx_v.at[j]], ssems[b],
                                 add=True)
                jn = j + LOOK
                bn = (u + LOOK) % NBUF

                @pl.when(jn < SCH)
                def _():
                    @pl.when(jn >= NBUF)
                    def _():
                        pltpu.make_async_copy(
                            bufs[bn], acc.at[cidx_v.at[jn - NBUF]],
                            ssems[bn]).wait()

                    pltpu.async_copy(g_hbm.at[ridx_v.at[jn]], bufs[bn],
                                     gsems[bn])

            return carry

        lax.fori_loop(0, SCH // NBUF, inner, 0)
        for b in range(NBUF):
            pltpu.make_async_copy(
                bufs[b], acc.at[cidx_v.at[SCH - NBUF + b]], ssems[b]).wait()

    plsc.subcore_barrier()
    for k in range(RPT // CH):
        pltpu.sync_copy(acc.at[pl.ds(s * RPT + k * CH, CH)], b0)
        pltpu.sync_copy(b0, out_hbm.at[c, pl.ds(s * RPT + k * CH, CH)])


def _agg(g, ridx, cidx):
    return pl.kernel(
        _agg_body,
        out_type=jax.ShapeDtypeStruct((NC, NP, NFE), jnp.float32),
        mesh=_mesh(),
        scratch_types=[
            pltpu.VMEM((SCH, CH), jnp.int32),
            pltpu.VMEM((SCH, CH), jnp.int32),
            pltpu.VMEM((CH, NFE), jnp.float32),
            pltpu.VMEM((CH, NFE), jnp.float32),
            pltpu.VMEM((CH, NFE), jnp.float32),
            pltpu.VMEM((CH, NFE), jnp.float32),
            pltpu.VMEM((CH, NFE), jnp.float32),
            pltpu.VMEM((CH, NFE), jnp.float32),
            pltpu.VMEM((CH, NFE), jnp.float32),
            pltpu.VMEM((CH, NFE), jnp.float32),
            pltpu.VMEM_SHARED((NP, NFE), jnp.float32),
            pltpu.SemaphoreType.DMA,
            pltpu.SemaphoreType.DMA,
            pltpu.SemaphoreType.DMA,
            pltpu.SemaphoreType.DMA,
            pltpu.SemaphoreType.DMA,
            pltpu.SemaphoreType.DMA,
            pltpu.SemaphoreType.DMA,
            pltpu.SemaphoreType.DMA,
            pltpu.SemaphoreType.DMA,
            pltpu.SemaphoreType.DMA,
            pltpu.SemaphoreType.DMA,
            pltpu.SemaphoreType.DMA,
            pltpu.SemaphoreType.DMA,
            pltpu.SemaphoreType.DMA,
            pltpu.SemaphoreType.DMA,
            pltpu.SemaphoreType.DMA,
        ],
    )(g, ridx, cidx)


def _dinv_of(d_ref):
    return lax.rsqrt(d_ref[0] + d_ref[1] + 1.0)


def _mm1_body(x_ref, w_ref, b_ref, d_ref, g_ref):
    i = pl.program_id(0)
    h = jnp.dot(x_ref[...], w_ref[...], preferred_element_type=jnp.float32)
    h = h + b_ref[...]
    dinv = _dinv_of(d_ref)
    rows = i * ROWB + lax.broadcasted_iota(jnp.int32, (ROWB, 1), 0)
    g_ref[...] = jnp.where(rows < NNODE, h * dinv[:, None], 0.0)


def _mm2_body(p_ref, g1_ref, d_ref, w_ref, b_ref, g2_ref):
    i = pl.program_id(0)
    dinv = _dinv_of(d_ref)
    ssum = p_ref[0] + p_ref[1] + g1_ref[...]
    h2 = jnp.maximum(dinv[:, None] * ssum, 0.0)
    o = jnp.dot(h2, w_ref[...], preferred_element_type=jnp.float32)
    o = o + b_ref[...]
    rows = i * ROWB + lax.broadcasted_iota(jnp.int32, (ROWB, 1), 0)
    g2_ref[:, :NCL] = jnp.where(rows < NNODE, o * dinv[:, None], 0.0)
    g2_ref[:, NCL:] = jnp.zeros((ROWB, NFE - NCL), jnp.float32)


def _out_body(p_ref, g2_ref, d_ref, o_ref):
    dinv = _dinv_of(d_ref)
    z = dinv[:, None] * (p_ref[0, :, :NCL] + p_ref[1, :, :NCL]
                         + g2_ref[:, :NCL])
    m = jnp.max(z, axis=1, keepdims=True)
    lse = jnp.log(jnp.sum(jnp.exp(z - m), axis=1, keepdims=True))
    o_ref[...] = z - m - lse


def _mm1(x, W1, b1, d):
    return pl.pallas_call(
        _mm1_body,
        grid=(GRID,),
        in_specs=[
            pl.BlockSpec((ROWB, NFE), lambda i: (i, 0)),
            pl.BlockSpec((NFE, NFE), lambda i: (0, 0)),
            pl.BlockSpec((1, NFE), lambda i: (0, 0)),
            pl.BlockSpec((NC, ROWB), lambda i: (0, i)),
        ],
        out_specs=pl.BlockSpec((ROWB, NFE), lambda i: (i, 0)),
        out_shape=jax.ShapeDtypeStruct((NP, NFE), jnp.float32),
    )(x, W1, b1.reshape(1, NFE), d)


def _mm2(p1, g1, d, W2, b2):
    return pl.pallas_call(
        _mm2_body,
        grid=(GRID,),
        in_specs=[
            pl.BlockSpec((NC, ROWB, NFE), lambda i: (0, i, 0)),
            pl.BlockSpec((ROWB, NFE), lambda i: (i, 0)),
            pl.BlockSpec((NC, ROWB), lambda i: (0, i)),
            pl.BlockSpec((NFE, NCL), lambda i: (0, 0)),
            pl.BlockSpec((1, NCL), lambda i: (0, 0)),
        ],
        out_specs=pl.BlockSpec((ROWB, NFE), lambda i: (i, 0)),
        out_shape=jax.ShapeDtypeStruct((NP, NFE), jnp.float32),
    )(p1, g1, d, W2, b2.reshape(1, NCL))


def _out(p2, g2, d):
    return pl.pallas_call(
        _out_body,
        grid=(GRID,),
        in_specs=[
            pl.BlockSpec((NC, ROWB, NFE), lambda i: (0, i, 0)),
            pl.BlockSpec((ROWB, NFE), lambda i: (i, 0)),
            pl.BlockSpec((NC, ROWB), lambda i: (0, i)),
        ],
        out_specs=pl.BlockSpec((ROWB, NCL), lambda i: (i, 0)),
        out_shape=jax.ShapeDtypeStruct((NNODE, NCL), jnp.float32),
    )(p2, g2, d)


def kernel(x, edge_index, W1, b1, W2, b2):
    nw = NC * NS
    epw_real = NEDGE // nw
    pad = jnp.full((nw, EPW - epw_real), NNODE, jnp.int32)
    row = jnp.concatenate([edge_index[0].reshape(nw, epw_real), pad], axis=1)
    col = jnp.concatenate([edge_index[1].reshape(nw, epw_real), pad], axis=1)
    ridx = row.reshape(NC, NS, NCHW, CH)
    cidx = col.reshape(NC, NS, NCHW, CH)
    d = _deg(ridx)
    g1 = _mm1(x, W1, b1, d)
    p1 = _agg(g1, ridx, cidx)
    g2 = _mm2(p1, g1, d, W2, b2)
    p2 = _agg(g2, ridx, cidx)
    return _out(p2, g2, d)


# speedup vs baseline: 1.0300x; 1.0300x over previous
import functools

import jax
import jax.numpy as jnp
from jax import lax
from jax.experimental import pallas as pl
from jax.experimental.pallas import tpu as pltpu
from jax.experimental.pallas import tpu_sc as plsc

NNODE = 10000
NP = 10240
NEDGE = 320000
NFE = 128
NCL = 64
NC, NS, LANES = 2, 16, 16
CH = 64
NCHW = 160
EPW = NCHW * CH
EPAD = NC * NS * EPW
NBUF = 4
LOOK = 3
NSTG = 4
SCH = NCHW // NSTG
RPT = NP // NS
ROWB = 512
GRID = NP // ROWB

def _mesh():
    return plsc.VectorSubcoreMesh(
        core_axis_name="c", subcore_axis_name="s", num_cores=NC, num_subcores=NS)


def _deg_body(ridx_hbm, out_hbm, ridx_v, ones_v, stage_v, acc):
    c = lax.axis_index("c")
    s = lax.axis_index("s")
    for q in range(CH // LANES):
        ones_v[pl.ds(q * LANES, LANES)] = jnp.ones((LANES,), jnp.float32)
    for q in range(RPT // LANES):
        stage_v[pl.ds(q * LANES, LANES)] = jnp.zeros((LANES,), jnp.float32)
    pltpu.sync_copy(ridx_hbm.at[c, s], ridx_v)
    pltpu.sync_copy(stage_v, acc.at[pl.ds(s * RPT, RPT)])
    plsc.subcore_barrier()

    def chunk(j, carry):
        pltpu.sync_copy(ones_v, acc.at[ridx_v.at[j]], add=True)
        return carry

    lax.fori_loop(0, NCHW, chunk, 0)
    plsc.subcore_barrier()
    pltpu.sync_copy(acc.at[pl.ds(s * RPT, RPT)], stage_v)
    pltpu.sync_copy(stage_v, out_hbm.at[c, pl.ds(s * RPT, RPT)])


def _deg(ridx):
    return pl.kernel(
        _deg_body,
        out_type=jax.ShapeDtypeStruct((NC, NP), jnp.float32),
        mesh=_mesh(),
        scratch_types=[
            pltpu.VMEM((NCHW, CH), jnp.int32),
            pltpu.VMEM((CH,), jnp.float32),
            pltpu.VMEM((RPT,), jnp.float32),
            pltpu.VMEM_SHARED((NP,), jnp.float32),
        ],
    )(ridx)


def _agg_body(g_hbm, ridx_hbm, cidx_hbm, out_hbm,
              ridx_v, cidx_v, b0, b1, b2, b3, acc,
              g0, g1, g2, g3, s0, s1, s2, s3):
    bufs = (b0, b1, b2, b3)
    gsems = (g0, g1, g2, g3)
    ssems = (s0, s1, s2, s3)
    c = lax.axis_index("c")
    s = lax.axis_index("s")
    for r in range(CH):
        for q in range(NFE // LANES):
            b0[r, pl.ds(q * LANES, LANES)] = jnp.zeros((LANES,), jnp.float32)
    for k in range(RPT // CH):
        pltpu.sync_copy(b0, acc.at[pl.ds(s * RPT + k * CH, CH)])
    plsc.subcore_barrier()

    for h in range(NSTG):
        pltpu.sync_copy(ridx_hbm.at[c, s, pl.ds(h * SCH, SCH)], ridx_v)
        pltpu.sync_copy(cidx_hbm.at[c, s, pl.ds(h * SCH, SCH)], cidx_v)
        for b in range(LOOK):
            pltpu.async_copy(g_hbm.at[ridx_v.at[b]], bufs[b], gsems[b])

        def inner(i, carry):
            for u in range(NBUF):
                j = i * NBUF + u
                b = u
                pltpu.make_async_copy(
                    g_hbm.at[ridx_v.at[j]], bufs[b], gsems[b]).wait()
                pltpu.async_copy(bufs[b], acc.at[cidx_v.at[j]], ssems[b],
                                 add=True)
                jn = j + LOOK
                bn = (u + LOOK) % NBUF

                @pl.when(jn < SCH)
                def _():
                    @pl.when(jn >= NBUF)
                    def _():
                        pltpu.make_async_copy(
                            bufs[bn], acc.at[cidx_v.at[jn - NBUF]],
                            ssems[bn]).wait()

                    pltpu.async_copy(g_hbm.at[ridx_v.at[jn]], bufs[bn],
                                     gsems[bn])

            return carry

        lax.fori_loop(0, SCH // NBUF, inner, 0)
        for b in range(NBUF):
            pltpu.make_async_copy(
                bufs[b], acc.at[cidx_v.at[SCH - NBUF + b]], ssems[b]).wait()

    plsc.subcore_barrier()
    pltpu.sync_copy(acc.at[pl.ds(s * RPT, RPT)],
                    out_hbm.at[c, pl.ds(s * RPT, RPT)])


def _agg(g, ridx, cidx):
    return pl.kernel(
        _agg_body,
        out_type=jax.ShapeDtypeStruct((NC, NP, NFE), jnp.float32),
        mesh=_mesh(),
        scratch_types=[
            pltpu.VMEM((SCH, CH), jnp.int32),
            pltpu.VMEM((SCH, CH), jnp.int32),
            pltpu.VMEM((CH, NFE), jnp.float32),
            pltpu.VMEM((CH, NFE), jnp.float32),
            pltpu.VMEM((CH, NFE), jnp.float32),
            pltpu.VMEM((CH, NFE), jnp.float32),
            pltpu.VMEM_SHARED((NP, NFE), jnp.float32),
            pltpu.SemaphoreType.DMA,
            pltpu.SemaphoreType.DMA,
            pltpu.SemaphoreType.DMA,
            pltpu.SemaphoreType.DMA,
            pltpu.SemaphoreType.DMA,
            pltpu.SemaphoreType.DMA,
            pltpu.SemaphoreType.DMA,
            pltpu.SemaphoreType.DMA,
        ],
    )(g, ridx, cidx)


def _dinv_of(d_ref):
    return lax.rsqrt(d_ref[0] + d_ref[1] + 1.0)


def _mm1_body(x_ref, w_ref, b_ref, d_ref, g_ref):
    i = pl.program_id(0)
    h = jnp.dot(x_ref[...], w_ref[...], preferred_element_type=jnp.float32)
    h = h + b_ref[...]
    dinv = _dinv_of(d_ref)
    rows = i * ROWB + lax.broadcasted_iota(jnp.int32, (ROWB, 1), 0)
    g_ref[...] = jnp.where(rows < NNODE, h * dinv[:, None], 0.0)


def _mm2_body(p_ref, g1_ref, d_ref, w_ref, b_ref, g2_ref):
    i = pl.program_id(0)
    dinv = _dinv_of(d_ref)
    ssum = p_ref[0] + p_ref[1] + g1_ref[...]
    h2 = jnp.maximum(dinv[:, None] * ssum, 0.0)
    o = jnp.dot(h2, w_ref[...], preferred_element_type=jnp.float32)
    o = o + b_ref[...]
    rows = i * ROWB + lax.broadcasted_iota(jnp.int32, (ROWB, 1), 0)
    g2_ref[:, :NCL] = jnp.where(rows < NNODE, o * dinv[:, None], 0.0)
    g2_ref[:, NCL:] = jnp.zeros((ROWB, NFE - NCL), jnp.float32)


def _out_body(p_ref, g2_ref, d_ref, o_ref):
    dinv = _dinv_of(d_ref)
    z = dinv[:, None] * (p_ref[0, :, :NCL] + p_ref[1, :, :NCL]
                         + g2_ref[:, :NCL])
    m = jnp.max(z, axis=1, keepdims=True)
    lse = jnp.log(jnp.sum(jnp.exp(z - m), axis=1, keepdims=True))
    o_ref[...] = z - m - lse


def _mm1(x, W1, b1, d):
    return pl.pallas_call(
        _mm1_body,
        grid=(GRID,),
        in_specs=[
            pl.BlockSpec((ROWB, NFE), lambda i: (i, 0)),
            pl.BlockSpec((NFE, NFE), lambda i: (0, 0)),
            pl.BlockSpec((1, NFE), lambda i: (0, 0)),
            pl.BlockSpec((NC, ROWB), lambda i: (0, i)),
        ],
        out_specs=pl.BlockSpec((ROWB, NFE), lambda i: (i, 0)),
        out_shape=jax.ShapeDtypeStruct((NP, NFE), jnp.float32),
    )(x, W1, b1.reshape(1, NFE), d)


def _mm2(p1, g1, d, W2, b2):
    return pl.pallas_call(
        _mm2_body,
        grid=(GRID,),
        in_specs=[
            pl.BlockSpec((NC, ROWB, NFE), lambda i: (0, i, 0)),
            pl.BlockSpec((ROWB, NFE), lambda i: (i, 0)),
            pl.BlockSpec((NC, ROWB), lambda i: (0, i)),
            pl.BlockSpec((NFE, NCL), lambda i: (0, 0)),
            pl.BlockSpec((1, NCL), lambda i: (0, 0)),
        ],
        out_specs=pl.BlockSpec((ROWB, NFE), lambda i: (i, 0)),
        out_shape=jax.ShapeDtypeStruct((NP, NFE), jnp.float32),
    )(p1, g1, d, W2, b2.reshape(1, NCL))


def _out(p2, g2, d):
    return pl.pallas_call(
        _out_body,
        grid=(GRID,),
        in_specs=[
            pl.BlockSpec((NC, ROWB, NFE), lambda i: (0, i, 0)),
            pl.BlockSpec((ROWB, NFE), lambda i: (i, 0)),
            pl.BlockSpec((NC, ROWB), lambda i: (0, i)),
        ],
        out_specs=pl.BlockSpec((ROWB, NCL), lambda i: (i, 0)),
        out_shape=jax.ShapeDtypeStruct((NNODE, NCL), jnp.float32),
    )(p2, g2, d)


def kernel(x, edge_index, W1, b1, W2, b2):
    nw = NC * NS
    epw_real = NEDGE // nw
    pad = jnp.full((nw, EPW - epw_real), NNODE, jnp.int32)
    row = jnp.concatenate([edge_index[0].reshape(nw, epw_real), pad], axis=1)
    col = jnp.concatenate([edge_index[1].reshape(nw, epw_real), pad], axis=1)
    ridx = row.reshape(NC, NS, NCHW, CH)
    cidx = col.reshape(NC, NS, NCHW, CH)
    d = _deg(ridx)
    g1 = _mm1(x, W1, b1, d)
    p1 = _agg(g1, ridx, cidx)
    g2 = _mm2(p1, g1, d, W2, b2)
    p2 = _agg(g2, ridx, cidx)
    return _out(p2, g2, d)


# speedup vs baseline: 1.0337x; 1.0035x over previous
import functools

import jax
import jax.numpy as jnp
from jax import lax
from jax.experimental import pallas as pl
from jax.experimental.pallas import tpu as pltpu
from jax.experimental.pallas import tpu_sc as plsc

NNODE = 10000
NP = 10240
NEDGE = 320000
NFE = 128
NCL = 64
NC, NS, LANES = 2, 16, 16
CH = 80
NCHW = 128
EPW = NCHW * CH
EPAD = NC * NS * EPW
NBUF = 4
LOOK = 3
NSTG = 4
SCH = NCHW // NSTG
RPT = NP // NS
ROWB = 512
GRID = NP // ROWB

def _mesh():
    return plsc.VectorSubcoreMesh(
        core_axis_name="c", subcore_axis_name="s", num_cores=NC, num_subcores=NS)


def _deg_body(ridx_hbm, out_hbm, ridx_v, ones_v, stage_v, acc):
    c = lax.axis_index("c")
    s = lax.axis_index("s")
    for q in range(CH // LANES):
        ones_v[pl.ds(q * LANES, LANES)] = jnp.ones((LANES,), jnp.float32)
    for q in range(RPT // LANES):
        stage_v[pl.ds(q * LANES, LANES)] = jnp.zeros((LANES,), jnp.float32)
    pltpu.sync_copy(ridx_hbm.at[c, s], ridx_v)
    pltpu.sync_copy(stage_v, acc.at[pl.ds(s * RPT, RPT)])
    plsc.subcore_barrier()

    def chunk(j, carry):
        pltpu.sync_copy(ones_v, acc.at[ridx_v.at[j]], add=True)
        return carry

    lax.fori_loop(0, NCHW, chunk, 0)
    plsc.subcore_barrier()
    pltpu.sync_copy(acc.at[pl.ds(s * RPT, RPT)], stage_v)
    pltpu.sync_copy(stage_v, out_hbm.at[c, pl.ds(s * RPT, RPT)])


def _deg(ridx):
    return pl.kernel(
        _deg_body,
        out_type=jax.ShapeDtypeStruct((NC, NP), jnp.float32),
        mesh=_mesh(),
        scratch_types=[
            pltpu.VMEM((NCHW, CH), jnp.int32),
            pltpu.VMEM((CH,), jnp.float32),
            pltpu.VMEM((RPT,), jnp.float32),
            pltpu.VMEM_SHARED((NP,), jnp.float32),
        ],
    )(ridx)


def _agg_body(g_hbm, ridx_hbm, cidx_hbm, out_hbm,
              ridx_v, cidx_v, b0, b1, b2, b3, acc,
              g0, g1, g2, g3, s0, s1, s2, s3):
    bufs = (b0, b1, b2, b3)
    gsems = (g0, g1, g2, g3)
    ssems = (s0, s1, s2, s3)
    c = lax.axis_index("c")
    s = lax.axis_index("s")
    for r in range(CH):
        for q in range(NFE // LANES):
            b0[r, pl.ds(q * LANES, LANES)] = jnp.zeros((LANES,), jnp.float32)
    for k in range(RPT // CH):
        pltpu.sync_copy(b0, acc.at[pl.ds(s * RPT + k * CH, CH)])
    plsc.subcore_barrier()

    for h in range(NSTG):
        pltpu.sync_copy(ridx_hbm.at[c, s, pl.ds(h * SCH, SCH)], ridx_v)
        pltpu.sync_copy(cidx_hbm.at[c, s, pl.ds(h * SCH, SCH)], cidx_v)
        for b in range(LOOK):
            pltpu.async_copy(g_hbm.at[ridx_v.at[b]], bufs[b], gsems[b])

        def inner(i, carry):
            for u in range(NBUF):
                j = i * NBUF + u
                b = u
                pltpu.make_async_copy(
                    g_hbm.at[ridx_v.at[j]], bufs[b], gsems[b]).wait()
                pltpu.async_copy(bufs[b], acc.at[cidx_v.at[j]], ssems[b],
                                 add=True)
                jn = j + LOOK
                bn = (u + LOOK) % NBUF

                @pl.when(jn < SCH)
                def _():
                    @pl.when(jn >= NBUF)
                    def _():
                        pltpu.make_async_copy(
                            bufs[bn], acc.at[cidx_v.at[jn - NBUF]],
                            ssems[bn]).wait()

                    pltpu.async_copy(g_hbm.at[ridx_v.at[jn]], bufs[bn],
                                     gsems[bn])

            return carry

        lax.fori_loop(0, SCH // NBUF, inner, 0)
        for b in range(NBUF):
            pltpu.make_async_copy(
                bufs[b], acc.at[cidx_v.at[SCH - NBUF + b]], ssems[b]).wait()

    plsc.subcore_barrier()
    pltpu.sync_copy(acc.at[pl.ds(s * RPT, RPT)],
                    out_hbm.at[c, pl.ds(s * RPT, RPT)])


def _agg(g, ridx, cidx):
    return pl.kernel(
        _agg_body,
        out_type=jax.ShapeDtypeStruct((NC, NP, NFE), jnp.float32),
        mesh=_mesh(),
        scratch_types=[
            pltpu.VMEM((SCH, CH), jnp.int32),
            pltpu.VMEM((SCH, CH), jnp.int32),
            pltpu.VMEM((CH, NFE), jnp.float32),
            pltpu.VMEM((CH, NFE), jnp.float32),
            pltpu.VMEM((CH, NFE), jnp.float32),
            pltpu.VMEM((CH, NFE), jnp.float32),
            pltpu.VMEM_SHARED((NP, NFE), jnp.float32),
            pltpu.SemaphoreType.DMA,
            pltpu.SemaphoreType.DMA,
            pltpu.SemaphoreType.DMA,
            pltpu.SemaphoreType.DMA,
            pltpu.SemaphoreType.DMA,
            pltpu.SemaphoreType.DMA,
            pltpu.SemaphoreType.DMA,
            pltpu.SemaphoreType.DMA,
        ],
    )(g, ridx, cidx)


def _dinv_of(d_ref):
    return lax.rsqrt(d_ref[0] + d_ref[1] + 1.0)


def _mm1_body(x_ref, w_ref, b_ref, d_ref, g_ref):
    i = pl.program_id(0)
    h = jnp.dot(x_ref[...], w_ref[...], preferred_element_type=jnp.float32)
    h = h + b_ref[...]
    dinv = _dinv_of(d_ref)
    rows = i * ROWB + lax.broadcasted_iota(jnp.int32, (ROWB, 1), 0)
    g_ref[...] = jnp.where(rows < NNODE, h * dinv[:, None], 0.0)


def _mm2_body(p_ref, g1_ref, d_ref, w_ref, b_ref, g2_ref):
    i = pl.program_id(0)
    dinv = _dinv_of(d_ref)
    ssum = p_ref[0] + p_ref[1] + g1_ref[...]
    h2 = jnp.maximum(dinv[:, None] * ssum, 0.0)
    o = jnp.dot(h2, w_ref[...], preferred_element_type=jnp.float32)
    o = o + b_ref[...]
    rows = i * ROWB + lax.broadcasted_iota(jnp.int32, (ROWB, 1), 0)
    g2_ref[:, :NCL] = jnp.where(rows < NNODE, o * dinv[:, None], 0.0)
    g2_ref[:, NCL:] = jnp.zeros((ROWB, NFE - NCL), jnp.float32)


def _out_body(p_ref, g2_ref, d_ref, o_ref):
    dinv = _dinv_of(d_ref)
    z = dinv[:, None] * (p_ref[0, :, :NCL] + p_ref[1, :, :NCL]
                         + g2_ref[:, :NCL])
    m = jnp.max(z, axis=1, keepdims=True)
    lse = jnp.log(jnp.sum(jnp.exp(z - m), axis=1, keepdims=True))
    o_ref[...] = z - m - lse


def _mm1(x, W1, b1, d):
    return pl.pallas_call(
        _mm1_body,
        grid=(GRID,),
        in_specs=[
            pl.BlockSpec((ROWB, NFE), lambda i: (i, 0)),
            pl.BlockSpec((NFE, NFE), lambda i: (0, 0)),
            pl.BlockSpec((1, NFE), lambda i: (0, 0)),
            pl.BlockSpec((NC, ROWB), lambda i: (0, i)),
        ],
        out_specs=pl.BlockSpec((ROWB, NFE), lambda i: (i, 0)),
        out_shape=jax.ShapeDtypeStruct((NP, NFE), jnp.float32),
    )(x, W1, b1.reshape(1, NFE), d)


def _mm2(p1, g1, d, W2, b2):
    return pl.pallas_call(
        _mm2_body,
        grid=(GRID,),
        in_specs=[
            pl.BlockSpec((NC, ROWB, NFE), lambda i: (0, i, 0)),
            pl.BlockSpec((ROWB, NFE), lambda i: (i, 0)),
            pl.BlockSpec((NC, ROWB), lambda i: (0, i)),
            pl.BlockSpec((NFE, NCL), lambda i: (0, 0)),
            pl.BlockSpec((1, NCL), lambda i: (0, 0)),
        ],
        out_specs=pl.BlockSpec((ROWB, NFE), lambda i: (i, 0)),
        out_shape=jax.ShapeDtypeStruct((NP, NFE), jnp.float32),
    )(p1, g1, d, W2, b2.reshape(1, NCL))


def _out(p2, g2, d):
    return pl.pallas_call(
        _out_body,
        grid=(GRID,),
        in_specs=[
            pl.BlockSpec((NC, ROWB, NFE), lambda i: (0, i, 0)),
            pl.BlockSpec((ROWB, NFE), lambda i: (i, 0)),
            pl.BlockSpec((NC, ROWB), lambda i: (0, i)),
        ],
        out_specs=pl.BlockSpec((ROWB, NCL), lambda i: (i, 0)),
        out_shape=jax.ShapeDtypeStruct((NNODE, NCL), jnp.float32),
    )(p2, g2, d)


def kernel(x, edge_index, W1, b1, W2, b2):
    nw = NC * NS
    epw_real = NEDGE // nw
    pad = jnp.full((nw, EPW - epw_real), NNODE, jnp.int32)
    row = jnp.concatenate([edge_index[0].reshape(nw, epw_real), pad], axis=1)
    col = jnp.concatenate([edge_index[1].reshape(nw, epw_real), pad], axis=1)
    ridx = row.reshape(NC, NS, NCHW, CH)
    cidx = col.reshape(NC, NS, NCHW, CH)
    d = _deg(ridx)
    g1 = _mm1(x, W1, b1, d)
    p1 = _agg(g1, ridx, cidx)
    g2 = _mm2(p1, g1, d, W2, b2)
    p2 = _agg(g2, ridx, cidx)
    return _out(p2, g2, d)
